# baseline (speedup 1.0000x reference)
import functools

import jax
import jax.numpy as jnp
from jax.experimental import pallas as pl
from jax.experimental.pallas import tpu as pltpu
from jax.experimental.pallas import tpu_sc as plsc

CHUNK = 128
TC_ROWS = 2048


def _coord_body(coords_ref, w_ref, b_ref, tt_ref, out_ref):
    bias = b_ref[0:1, :] + tt_ref[1:2, :]
    out_ref[...] = (
        jnp.dot(coords_ref[...], w_ref[...], preferred_element_type=jnp.float32)
        + bias
    )


def _coord_linear(coords2d, W, b2d, type_table, rows, d):
    return pl.pallas_call(
        _coord_body,
        grid=(rows // TC_ROWS,),
        in_specs=[
            pl.BlockSpec((TC_ROWS, 2), lambda i: (i, 0)),
            pl.BlockSpec((2, d), lambda i: (0, 0)),
            pl.BlockSpec((1, d), lambda i: (0, 0)),
            pl.BlockSpec((2, d), lambda i: (0, 0)),
        ],
        out_specs=pl.BlockSpec((TC_ROWS, d), lambda i: (i, 0)),
        out_shape=jax.ShapeDtypeStruct((rows, d), jnp.float32),
    )(coords2d, W, b2d, type_table)


def _sc_special(special_table, type_table, ids2d, rows, d):
    mesh = plsc.VectorSubcoreMesh(
        core_axis_name="core", subcore_axis_name="subcore"
    )

    @functools.partial(
        pl.kernel,
        out_type=jax.ShapeDtypeStruct((rows, d), jnp.float32),
        mesh=mesh,
        scratch_types=[pltpu.VMEM((d,), jnp.float32)],
    )
    def k(table_hbm, tt_hbm, ids_hbm, out_hbm, t0_vmem):
        pltpu.sync_copy(tt_hbm.at[0], t0_vmem)

        def body(i_vmem, o_vmem):
            pltpu.sync_copy(table_hbm.at[i_vmem.at[0]], o_vmem)

            @pl.loop(0, CHUNK)
            def _(r):
                for c in range(0, d, 16):
                    o_vmem[r, pl.ds(c, 16)] = (
                        o_vmem[r, pl.ds(c, 16)] + t0_vmem[pl.ds(c, 16)]
                    )

        pltpu.emit_pipeline(
            body,
            grid=(rows // CHUNK,),
            in_specs=[pl.BlockSpec((1, CHUNK), lambda i: (0, i))],
            out_specs=[pl.BlockSpec((CHUNK, d), lambda i: (i, 0))],
            core_axis_name=("core", "subcore"),
            dimension_semantics=(pltpu.PARALLEL,),
        )(ids_hbm, out_hbm)

    return k(special_table, type_table, ids2d)


def kernel(token_ids, coords, special_table, type_table, W, b):
    bsz, t = token_ids.shape
    d = special_table.shape[1]
    rows = bsz * t

    ids2d = token_ids.reshape(1, rows).astype(jnp.int32)
    coords2d = coords.reshape(rows, 2)
    b2d = b.reshape(1, d)

    special_flat = _sc_special(special_table, type_table, ids2d, rows, d)
    coord_flat = _coord_linear(coords2d, W, b2d, type_table, rows, d)
    return special_flat.reshape(bsz, t, d), coord_flat.reshape(bsz, t, d)


# speedup vs baseline: 1.0020x; 1.0020x over previous
import functools

import jax
import jax.numpy as jnp
from jax import lax
from jax.experimental import pallas as pl
from jax.experimental.pallas import tpu as pltpu
from jax.experimental.pallas import tpu_sc as plsc

NUM_CORES = 2
NUM_SUBCORES = 16
NUM_WORKERS = NUM_CORES * NUM_SUBCORES
CHUNK = 128
TC_ROWS = 2048


def _prep_body(sp_ref, tt_ref, comb_ref):
    comb_ref[...] = sp_ref[...] + tt_ref[0:1, :]


def _combined_table(special_table, type_table):
    n, d = special_table.shape
    return pl.pallas_call(
        _prep_body,
        out_shape=jax.ShapeDtypeStruct((n, d), jnp.float32),
    )(special_table, type_table)


def _coord_body(coords_ref, w_ref, b_ref, tt_ref, out_ref):
    bias = b_ref[0:1, :] + tt_ref[1:2, :]
    out_ref[...] = (
        jnp.dot(coords_ref[...], w_ref[...], preferred_element_type=jnp.float32)
        + bias
    )


def _coord_linear(coords2d, W, b2d, type_table, rows, d):
    return pl.pallas_call(
        _coord_body,
        grid=(rows // TC_ROWS,),
        in_specs=[
            pl.BlockSpec((TC_ROWS, 2), lambda i: (i, 0)),
            pl.BlockSpec((2, d), lambda i: (0, 0)),
            pl.BlockSpec((1, d), lambda i: (0, 0)),
            pl.BlockSpec((2, d), lambda i: (0, 0)),
        ],
        out_specs=pl.BlockSpec((TC_ROWS, d), lambda i: (i, 0)),
        out_shape=jax.ShapeDtypeStruct((rows, d), jnp.float32),
    )(coords2d, W, b2d, type_table)


def _sc_special(comb_table, ids_flat, rows, d):
    per_w = rows // NUM_WORKERS
    nchunk = per_w // CHUNK
    mesh = plsc.VectorSubcoreMesh(
        core_axis_name="core", subcore_axis_name="subcore"
    )

    @functools.partial(
        pl.kernel,
        out_type=jax.ShapeDtypeStruct((rows, d), jnp.float32),
        mesh=mesh,
        scratch_types=[
            pltpu.VMEM((per_w,), jnp.int32),
            pltpu.VMEM((CHUNK, d), jnp.float32),
            pltpu.VMEM((CHUNK, d), jnp.float32),
            pltpu.SemaphoreType.DMA,
            pltpu.SemaphoreType.DMA,
            pltpu.SemaphoreType.DMA,
            pltpu.SemaphoreType.DMA,
        ],
    )
    def k(table_hbm, ids_hbm, out_hbm, ids_v, buf0, buf1, g0, g1, w0, w1):
        wid = lax.axis_index("subcore") * NUM_CORES + lax.axis_index("core")
        base = wid * per_w
        pltpu.sync_copy(ids_hbm.at[pl.ds(base, per_w)], ids_v)

        bufs = (buf0, buf1)
        gsem = (g0, g1)
        wsem = (w0, w1)
        gathers = [None] * nchunk
        writes = [None] * nchunk
        for i in range(nchunk):
            s = i % 2
            if i >= 2:
                writes[i - 2].wait()
            gathers[i] = pltpu.async_copy(
                table_hbm.at[ids_v.at[pl.ds(i * CHUNK, CHUNK)]],
                bufs[s],
                gsem[s],
            )
            if i >= 1:
                t = (i - 1) % 2
                gathers[i - 1].wait()
                writes[i - 1] = pltpu.async_copy(
                    bufs[t],
                    out_hbm.at[pl.ds(base + (i - 1) * CHUNK, CHUNK)],
                    wsem[t],
                )
        gathers[nchunk - 1].wait()
        writes[nchunk - 1] = pltpu.async_copy(
            bufs[(nchunk - 1) % 2],
            out_hbm.at[pl.ds(base + (nchunk - 1) * CHUNK, CHUNK)],
            wsem[(nchunk - 1) % 2],
        )
        writes[nchunk - 2].wait()
        writes[nchunk - 1].wait()

    return k(comb_table, ids_flat)


def kernel(token_ids, coords, special_table, type_table, W, b):
    bsz, t = token_ids.shape
    d = special_table.shape[1]
    rows = bsz * t

    ids_flat = token_ids.reshape(rows).astype(jnp.int32)
    coords2d = coords.reshape(rows, 2)
    b2d = b.reshape(1, d)

    comb = _combined_table(special_table, type_table)
    special_flat = _sc_special(comb, ids_flat, rows, d)
    coord_flat = _coord_linear(coords2d, W, b2d, type_table, rows, d)
    return special_flat.reshape(bsz, t, d), coord_flat.reshape(bsz, t, d)


# speedup vs baseline: 1.0030x; 1.0010x over previous
import functools

import jax
import jax.numpy as jnp
from jax import lax
from jax.experimental import pallas as pl
from jax.experimental.pallas import tpu as pltpu
from jax.experimental.pallas import tpu_sc as plsc

NUM_CORES = 2
NUM_SUBCORES = 16
NUM_WORKERS = NUM_CORES * NUM_SUBCORES
CHUNK = 128
TC_ROWS = 2048


def _prep_body(sp_ref, tt_ref, comb_ref):
    comb_ref[...] = sp_ref[...] + tt_ref[0:1, :]


def _combined_table(special_table, type_table):
    n, d = special_table.shape
    return pl.pallas_call(
        _prep_body,
        out_shape=jax.ShapeDtypeStruct((n, d), jnp.float32),
    )(special_table, type_table)


def _coord_body(coords_ref, w_ref, b_ref, tt_ref, out_ref):
    bias = b_ref[0:1, :] + tt_ref[1:2, :]
    out_ref[...] = (
        jnp.dot(coords_ref[...], w_ref[...], preferred_element_type=jnp.float32)
        + bias
    )


def _coord_linear(coords2d, W, b2d, type_table, rows, d):
    return pl.pallas_call(
        _coord_body,
        grid=(rows // TC_ROWS,),
        in_specs=[
            pl.BlockSpec((TC_ROWS, 2), lambda i: (i, 0)),
            pl.BlockSpec((2, d), lambda i: (0, 0)),
            pl.BlockSpec((1, d), lambda i: (0, 0)),
            pl.BlockSpec((2, d), lambda i: (0, 0)),
        ],
        out_specs=pl.BlockSpec((TC_ROWS, d), lambda i: (i, 0)),
        out_shape=jax.ShapeDtypeStruct((rows, d), jnp.float32),
    )(coords2d, W, b2d, type_table)


def _sc_special(comb_table, ids2d, rows, d):
    per_w = rows // NUM_WORKERS
    nchunk = per_w // CHUNK
    mesh = plsc.VectorSubcoreMesh(
        core_axis_name="core", subcore_axis_name="subcore"
    )

    @functools.partial(
        pl.kernel,
        out_type=jax.ShapeDtypeStruct((rows, d), jnp.float32),
        mesh=mesh,
        scratch_types=[
            pltpu.VMEM((nchunk, CHUNK), jnp.int32),
            pltpu.VMEM((CHUNK, d), jnp.float32),
            pltpu.VMEM((CHUNK, d), jnp.float32),
            pltpu.SemaphoreType.DMA,
            pltpu.SemaphoreType.DMA,
            pltpu.SemaphoreType.DMA,
            pltpu.SemaphoreType.DMA,
        ],
    )
    def k(table_hbm, ids_hbm, out_hbm, ids_v, buf0, buf1, g0, g1, w0, w1):
        cid = lax.axis_index("core")
        sid = lax.axis_index("subcore")
        wid = sid * NUM_CORES + cid
        base = wid * per_w
        pltpu.sync_copy(ids_hbm.at[wid], ids_v)

        bufs = (buf0, buf1)
        gsem = (g0, g1)
        wsem = (w0, w1)
        gathers = [None] * nchunk
        writes = [None] * nchunk
        for i in range(nchunk):
            s = i % 2
            if i >= 2:
                writes[i - 2].wait()
            gathers[i] = pltpu.async_copy(
                table_hbm.at[ids_v.at[i]],
                bufs[s],
                gsem[s],
            )
            if i >= 1:
                t = (i - 1) % 2
                gathers[i - 1].wait()
                writes[i - 1] = pltpu.async_copy(
                    bufs[t],
                    out_hbm.at[pl.ds(base + (i - 1) * CHUNK, CHUNK)],
                    wsem[t],
                )
        gathers[nchunk - 1].wait()
        writes[nchunk - 1] = pltpu.async_copy(
            bufs[(nchunk - 1) % 2],
            out_hbm.at[pl.ds(base + (nchunk - 1) * CHUNK, CHUNK)],
            wsem[(nchunk - 1) % 2],
        )
        writes[nchunk - 2].wait()
        writes[nchunk - 1].wait()

    return k(comb_table, ids2d)


def kernel(token_ids, coords, special_table, type_table, W, b):
    bsz, t = token_ids.shape
    d = special_table.shape[1]
    rows = bsz * t

    ids2d = token_ids.reshape(
        NUM_WORKERS, rows // (NUM_WORKERS * CHUNK), CHUNK
    ).astype(jnp.int32)
    coords2d = coords.reshape(rows, 2)
    b2d = b.reshape(1, d)

    comb = _combined_table(special_table, type_table)
    special_flat = _sc_special(comb, ids2d, rows, d)
    coord_flat = _coord_linear(coords2d, W, b2d, type_table, rows, d)
    return special_flat.reshape(bsz, t, d), coord_flat.reshape(bsz, t, d)


# speedup vs baseline: 4.5054x; 4.4920x over previous
import dataclasses
import functools

import jax
import jax.numpy as jnp
from jax import lax
from jax.experimental import pallas as pl
from jax.experimental.pallas import tpu as pltpu
from jax.experimental.pallas import tpu_sc as plsc

NUM_CORES = 2
NUM_SUBCORES = 16
NUM_WORKERS = NUM_CORES * NUM_SUBCORES
CHUNK = 128
TC_ROWS = 2048


def _prep_body(sp_ref, tt_ref, comb_ref):
    comb_ref[...] = sp_ref[...] + tt_ref[0:1, :]


def _combined_table(special_table, type_table):
    n, d = special_table.shape
    return pl.pallas_call(
        _prep_body,
        out_shape=jax.ShapeDtypeStruct((n, d), jnp.float32),
    )(special_table, type_table)


def _coord_body(coords_ref, w_ref, b_ref, tt_ref, out_ref):
    bias = b_ref[0:1, :] + tt_ref[1:2, :]
    out_ref[...] = (
        jnp.dot(coords_ref[...], w_ref[...], preferred_element_type=jnp.float32)
        + bias
    )


def _coord_linear(coords2d, W, b2d, type_table, rows, d):
    return pl.pallas_call(
        _coord_body,
        grid=(rows // TC_ROWS,),
        in_specs=[
            pl.BlockSpec((TC_ROWS, 2), lambda i: (i, 0)),
            pl.BlockSpec((2, d), lambda i: (0, 0)),
            pl.BlockSpec((1, d), lambda i: (0, 0)),
            pl.BlockSpec((2, d), lambda i: (0, 0)),
        ],
        out_specs=pl.BlockSpec((TC_ROWS, d), lambda i: (i, 0)),
        out_shape=jax.ShapeDtypeStruct((rows, d), jnp.float32),
    )(coords2d, W, b2d, type_table)


def _sc_special(comb_flat, ids3d, rows, d):
    per_w = rows // NUM_WORKERS
    nchunk = per_w // CHUNK
    mesh = plsc.VectorSubcoreMesh(
        core_axis_name="core", subcore_axis_name="subcore"
    )
    cp = pltpu.CompilerParams()
    if "needs_layout_passes" in pltpu.CompilerParams.__dataclass_fields__:
        cp = dataclasses.replace(cp, needs_layout_passes=False)

    @functools.partial(
        pl.kernel,
        out_type=jax.ShapeDtypeStruct((rows * d,), jnp.float32),
        mesh=mesh,
        compiler_params=cp,
        scratch_types=[
            pltpu.VMEM((nchunk, CHUNK), jnp.int32),
            pltpu.VMEM((CHUNK * d,), jnp.float32),
            pltpu.VMEM((CHUNK * d,), jnp.float32),
            pltpu.VMEM((3 * d,), jnp.float32),
            pltpu.SemaphoreType.DMA,
            pltpu.SemaphoreType.DMA,
        ],
    )
    def k(table_hbm, ids_hbm, out_hbm, ids_v, buf0, buf1, tab_v, w0, w1):
        wid = lax.axis_index("subcore") * NUM_CORES + lax.axis_index("core")
        base = wid * per_w
        pltpu.sync_copy(ids_hbm.at[wid], ids_v)
        pltpu.sync_copy(table_hbm, tab_v)

        lanes = lax.iota(jnp.int32, 16)
        bufs = (buf0, buf1)
        wsem = (w0, w1)

        @pl.loop(0, nchunk, step=2)
        def _(g):
            for b in range(2):
                i = g + b
                buf = bufs[b]

                @pl.when(i >= 2)
                def _():
                    pltpu.make_async_copy(
                        buf, out_hbm.at[pl.ds(0, CHUNK * d)], wsem[b]
                    ).wait()

                @pl.loop(0, CHUNK)
                def _(r, _buf=buf, _i=i):
                    tid = plsc.load_gather(
                        ids_v,
                        [jnp.full((16,), _i, jnp.int32),
                         jnp.full((16,), r, jnp.int32)],
                    )
                    row_off = tid * d + lanes
                    rb = r * d
                    for c in range(0, d, 16):
                        _buf[pl.ds(rb + c, 16)] = plsc.load_gather(
                            tab_v, [row_off + c]
                        )

                pltpu.async_copy(
                    buf,
                    out_hbm.at[pl.ds((base + i * CHUNK) * d, CHUNK * d)],
                    wsem[b],
                )

        for b in range(2):
            pltpu.make_async_copy(
                bufs[b], out_hbm.at[pl.ds(0, CHUNK * d)], wsem[b]
            ).wait()

    return k(comb_flat, ids3d)


def kernel(token_ids, coords, special_table, type_table, W, b):
    bsz, t = token_ids.shape
    d = special_table.shape[1]
    rows = bsz * t

    ids3d = token_ids.reshape(
        NUM_WORKERS, rows // (NUM_WORKERS * CHUNK), CHUNK
    ).astype(jnp.int32)
    coords2d = coords.reshape(rows, 2)
    b2d = b.reshape(1, d)

    comb = _combined_table(special_table, type_table)
    special_flat = _sc_special(comb.reshape(3 * d), ids3d, rows, d)
    coord_flat = _coord_linear(coords2d, W, b2d, type_table, rows, d)
    return special_flat.reshape(bsz, t, d), coord_flat.reshape(bsz, t, d)


# speedup vs baseline: 6.7158x; 1.4906x over previous
import dataclasses
import functools

import jax
import jax.numpy as jnp
from jax import lax
from jax.experimental import pallas as pl
from jax.experimental.pallas import tpu as pltpu
from jax.experimental.pallas import tpu_sc as plsc

NUM_CORES = 2
NUM_SUBCORES = 16
NUM_WORKERS = NUM_CORES * NUM_SUBCORES
CHUNK = 128
TC_ROWS = 2048


def _prep_body(sp_ref, tt_ref, comb_ref):
    comb_ref[...] = sp_ref[...] + tt_ref[0:1, :]


def _combined_table(special_table, type_table):
    n, d = special_table.shape
    return pl.pallas_call(
        _prep_body,
        out_shape=jax.ShapeDtypeStruct((n, d), jnp.float32),
    )(special_table, type_table)


def _coord_body(coords_ref, w_ref, b_ref, tt_ref, out_ref):
    bias = b_ref[0:1, :] + tt_ref[1:2, :]
    out_ref[...] = (
        jnp.dot(coords_ref[...], w_ref[...], preferred_element_type=jnp.float32)
        + bias
    )


def _coord_linear(coords2d, W, b2d, type_table, rows, d):
    return pl.pallas_call(
        _coord_body,
        grid=(rows // TC_ROWS,),
        in_specs=[
            pl.BlockSpec((TC_ROWS, 2), lambda i: (i, 0)),
            pl.BlockSpec((2, d), lambda i: (0, 0)),
            pl.BlockSpec((1, d), lambda i: (0, 0)),
            pl.BlockSpec((2, d), lambda i: (0, 0)),
        ],
        out_specs=pl.BlockSpec((TC_ROWS, d), lambda i: (i, 0)),
        out_shape=jax.ShapeDtypeStruct((rows, d), jnp.float32),
    )(coords2d, W, b2d, type_table)


def _sc_special(comb_flat, ids3d, rows, d):
    per_w = rows // NUM_WORKERS
    nchunk = per_w // CHUNK
    mesh = plsc.VectorSubcoreMesh(
        core_axis_name="core", subcore_axis_name="subcore"
    )
    cp = pltpu.CompilerParams()
    if "needs_layout_passes" in pltpu.CompilerParams.__dataclass_fields__:
        cp = dataclasses.replace(cp, needs_layout_passes=False)

    @functools.partial(
        pl.kernel,
        out_type=jax.ShapeDtypeStruct((rows * d,), jnp.float32),
        mesh=mesh,
        compiler_params=cp,
        scratch_types=[
            pltpu.VMEM((nchunk, CHUNK), jnp.int32),
            pltpu.VMEM((CHUNK * d,), jnp.float32),
            pltpu.VMEM((CHUNK * d,), jnp.float32),
            pltpu.VMEM((3 * d,), jnp.float32),
            pltpu.SemaphoreType.DMA,
            pltpu.SemaphoreType.DMA,
        ],
    )
    def k(table_hbm, ids_hbm, out_hbm, ids_v, buf0, buf1, tab_v, w0, w1):
        wid = lax.axis_index("subcore") * NUM_CORES + lax.axis_index("core")
        base = wid * per_w
        pltpu.sync_copy(ids_hbm.at[wid], ids_v)
        pltpu.sync_copy(table_hbm, tab_v)

        lanes = lax.iota(jnp.int32, 16)
        bufs = (buf0, buf1)
        wsem = (w0, w1)

        @pl.loop(0, nchunk, step=2)
        def _(g):
            for b in range(2):
                i = g + b
                buf = bufs[b]

                @pl.when(i >= 2)
                def _():
                    pltpu.make_async_copy(
                        buf, out_hbm.at[pl.ds(0, CHUNK * d)], wsem[b]
                    ).wait()

                @pl.loop(0, CHUNK, step=4)
                def _(r, _buf=buf, _i=i):
                    ci = jnp.full((16,), _i, jnp.int32)
                    offs = []
                    for u in range(4):
                        tid = plsc.load_gather(
                            ids_v, [ci, jnp.full((16,), r + u, jnp.int32)]
                        )
                        offs.append(tid * d + lanes)
                    rb = r * d
                    for c in range(0, d, 16):
                        vals = [
                            plsc.load_gather(tab_v, [offs[u] + c])
                            for u in range(4)
                        ]
                        for u in range(4):
                            _buf[pl.ds(rb + u * d + c, 16)] = vals[u]

                pltpu.async_copy(
                    buf,
                    out_hbm.at[pl.ds((base + i * CHUNK) * d, CHUNK * d)],
                    wsem[b],
                )

        for b in range(2):
            pltpu.make_async_copy(
                bufs[b], out_hbm.at[pl.ds(0, CHUNK * d)], wsem[b]
            ).wait()

    return k(comb_flat, ids3d)


def kernel(token_ids, coords, special_table, type_table, W, b):
    bsz, t = token_ids.shape
    d = special_table.shape[1]
    rows = bsz * t

    ids3d = token_ids.reshape(
        NUM_WORKERS, rows // (NUM_WORKERS * CHUNK), CHUNK
    ).astype(jnp.int32)
    coords2d = coords.reshape(rows, 2)
    b2d = b.reshape(1, d)

    comb = _combined_table(special_table, type_table)
    special_flat = _sc_special(comb.reshape(3 * d), ids3d, rows, d)
    coord_flat = _coord_linear(coords2d, W, b2d, type_table, rows, d)
    return special_flat.reshape(bsz, t, d), coord_flat.reshape(bsz, t, d)


# speedup vs baseline: 13.4993x; 2.0101x over previous
import dataclasses
import functools

import jax
import jax.numpy as jnp
from jax import lax
from jax.experimental import pallas as pl
from jax.experimental.pallas import tpu as pltpu
from jax.experimental.pallas import tpu_sc as plsc

NUM_CORES = 2
NUM_SUBCORES = 16
NUM_WORKERS = NUM_CORES * NUM_SUBCORES
TC_BATCH = 16


def _prep_body(sp_ref, tt_ref, comb_ref):
    comb_ref[...] = sp_ref[...] + tt_ref[0:1, :]


def _combined_table(special_table, type_table):
    n, d = special_table.shape
    return pl.pallas_call(
        _prep_body,
        out_shape=jax.ShapeDtypeStruct((n, d), jnp.float32),
    )(special_table, type_table)


def _coord_body(coords_ref, w_ref, b_ref, tt_ref, out_ref):
    bsz_blk, t, _ = coords_ref.shape
    d = w_ref.shape[1]
    bias = (b_ref[0:1, :] + tt_ref[1:2, :]).reshape(1, 1, d)
    out_ref[...] = (
        jax.lax.dot_general(
            coords_ref[...], w_ref[...],
            dimension_numbers=(((2,), (0,)), ((), ())),
            preferred_element_type=jnp.float32,
        )
        + bias
    )


def _coord_linear(coords, W, b2d, type_table):
    bsz, t, _ = coords.shape
    d = W.shape[1]
    return pl.pallas_call(
        _coord_body,
        grid=(bsz // TC_BATCH,),
        in_specs=[
            pl.BlockSpec((TC_BATCH, t, 2), lambda i: (i, 0, 0)),
            pl.BlockSpec((2, d), lambda i: (0, 0)),
            pl.BlockSpec((1, d), lambda i: (0, 0)),
            pl.BlockSpec((2, d), lambda i: (0, 0)),
        ],
        out_specs=pl.BlockSpec((TC_BATCH, t, d), lambda i: (i, 0, 0)),
        out_shape=jax.ShapeDtypeStruct((bsz, t, d), jnp.float32),
    )(coords, W, b2d, type_table)


def _sc_special(comb_flat, token_ids, d):
    bsz, t = token_ids.shape
    rows = bsz * t
    rows_w = bsz // NUM_WORKERS
    mesh = plsc.VectorSubcoreMesh(
        core_axis_name="core", subcore_axis_name="subcore"
    )
    cp = pltpu.CompilerParams()
    if "needs_layout_passes" in pltpu.CompilerParams.__dataclass_fields__:
        cp = dataclasses.replace(cp, needs_layout_passes=False)

    @functools.partial(
        pl.kernel,
        out_type=jax.ShapeDtypeStruct((rows, d), jnp.float32),
        mesh=mesh,
        compiler_params=cp,
        scratch_types=[
            pltpu.VMEM((rows_w, t), jnp.int32),
            pltpu.VMEM((t, d), jnp.float32),
            pltpu.VMEM((t, d), jnp.float32),
            pltpu.VMEM((3 * d,), jnp.float32),
            pltpu.SemaphoreType.DMA,
            pltpu.SemaphoreType.DMA,
        ],
    )
    def k(table_hbm, ids_hbm, out_hbm, ids_v, buf0, buf1, tab_v, w0, w1):
        wid = lax.axis_index("subcore") * NUM_CORES + lax.axis_index("core")
        pltpu.sync_copy(ids_hbm.at[pl.ds(wid * rows_w, rows_w)], ids_v)
        pltpu.sync_copy(table_hbm, tab_v)

        lanes = lax.iota(jnp.int32, 16)
        bufs = (buf0, buf1)
        wsem = (w0, w1)

        @pl.loop(0, rows_w, step=2)
        def _(g):
            for b in range(2):
                i = g + b
                buf = bufs[b]

                @pl.when(i >= 2)
                def _():
                    pltpu.make_async_copy(
                        buf, out_hbm.at[pl.ds(0, t)], wsem[b]
                    ).wait()

                @pl.loop(0, t, step=4)
                def _(r, _buf=buf, _i=i):
                    ci = jnp.full((16,), _i, jnp.int32)
                    offs = []
                    for u in range(4):
                        tid = plsc.load_gather(
                            ids_v, [ci, jnp.full((16,), r + u, jnp.int32)]
                        )
                        offs.append(tid * d + lanes)
                    for c in range(0, d, 16):
                        vals = [
                            plsc.load_gather(tab_v, [offs[u] + c])
                            for u in range(4)
                        ]
                        for u in range(4):
                            _buf[r + u, pl.ds(c, 16)] = vals[u]

                pltpu.async_copy(
                    buf,
                    out_hbm.at[pl.ds((wid * rows_w + i) * t, t)],
                    wsem[b],
                )

        for b in range(2):
            pltpu.make_async_copy(
                bufs[b], out_hbm.at[pl.ds(0, t)], wsem[b]
            ).wait()

    return k(comb_flat, token_ids)


def kernel(token_ids, coords, special_table, type_table, W, b):
    bsz, t = token_ids.shape
    d = special_table.shape[1]

    b2d = b.reshape(1, d)
    comb = _combined_table(special_table, type_table)
    special_flat = _sc_special(
        comb.reshape(3 * d), token_ids.astype(jnp.int32), d
    )
    coord_out = _coord_linear(coords, W, b2d, type_table)
    return special_flat.reshape(bsz, t, d), coord_out


# speedup vs baseline: 17.0433x; 1.2625x over previous
import dataclasses
import functools

import jax
import jax.numpy as jnp
from jax import lax
from jax.experimental import pallas as pl
from jax.experimental.pallas import tpu as pltpu
from jax.experimental.pallas import tpu_sc as plsc

NUM_CORES = 2
NUM_SUBCORES = 16
NUM_WORKERS = NUM_CORES * NUM_SUBCORES
TC_BATCH = 16


def _prep_body(sp_ref, tt_ref, comb_ref):
    comb_ref[...] = sp_ref[...] + tt_ref[0:1, :]


def _combined_table(special_table, type_table):
    n, d = special_table.shape
    return pl.pallas_call(
        _prep_body,
        out_shape=jax.ShapeDtypeStruct((n, d), jnp.float32),
    )(special_table, type_table)


def _coord_body(cx_ref, cy_ref, w_ref, b_ref, tt_ref, out_ref):
    blk_b, t = cx_ref.shape
    d = w_ref.shape[1]
    bias = (b_ref[0:1, :] + tt_ref[1:2, :]).reshape(1, 1, d)
    w0 = w_ref[0:1, :].reshape(1, 1, d)
    w1 = w_ref[1:2, :].reshape(1, 1, d)
    cx3 = jax.lax.broadcast_in_dim(cx_ref[...], (blk_b, t, d), (0, 1))
    cy3 = jax.lax.broadcast_in_dim(cy_ref[...], (blk_b, t, d), (0, 1))
    out_ref[...] = cx3 * w0 + cy3 * w1 + bias


def _coord_linear(cx, cy, W, b2d, type_table):
    bsz, t = cx.shape
    d = W.shape[1]
    return pl.pallas_call(
        _coord_body,
        grid=(bsz // TC_BATCH,),
        in_specs=[
            pl.BlockSpec((TC_BATCH, t), lambda i: (i, 0)),
            pl.BlockSpec((TC_BATCH, t), lambda i: (i, 0)),
            pl.BlockSpec((2, d), lambda i: (0, 0)),
            pl.BlockSpec((1, d), lambda i: (0, 0)),
            pl.BlockSpec((2, d), lambda i: (0, 0)),
        ],
        out_specs=pl.BlockSpec((TC_BATCH, t, d), lambda i: (i, 0, 0)),
        out_shape=jax.ShapeDtypeStruct((bsz, t, d), jnp.float32),
    )(cx, cy, W, b2d, type_table)


def _sc_special(comb_flat, token_ids, d):
    bsz, t = token_ids.shape
    rows = bsz * t
    rows_w = bsz // NUM_WORKERS
    mesh = plsc.VectorSubcoreMesh(
        core_axis_name="core", subcore_axis_name="subcore"
    )
    cp = pltpu.CompilerParams()
    if "needs_layout_passes" in pltpu.CompilerParams.__dataclass_fields__:
        cp = dataclasses.replace(cp, needs_layout_passes=False)

    @functools.partial(
        pl.kernel,
        out_type=jax.ShapeDtypeStruct((rows, d), jnp.float32),
        mesh=mesh,
        compiler_params=cp,
        scratch_types=[
            pltpu.VMEM((rows_w, t), jnp.int32),
            pltpu.VMEM((t, d), jnp.float32),
            pltpu.VMEM((t, d), jnp.float32),
            pltpu.VMEM((3 * d,), jnp.float32),
            pltpu.SemaphoreType.DMA,
            pltpu.SemaphoreType.DMA,
        ],
    )
    def k(table_hbm, ids_hbm, out_hbm, ids_v, buf0, buf1, tab_v, w0, w1):
        wid = lax.axis_index("subcore") * NUM_CORES + lax.axis_index("core")
        pltpu.sync_copy(ids_hbm.at[pl.ds(wid * rows_w, rows_w)], ids_v)
        pltpu.sync_copy(table_hbm, tab_v)

        lanes = lax.iota(jnp.int32, 16)
        bufs = (buf0, buf1)
        wsem = (w0, w1)

        @pl.loop(0, rows_w, step=2)
        def _(g):
            for b in range(2):
                i = g + b
                buf = bufs[b]

                @pl.when(i >= 2)
                def _():
                    pltpu.make_async_copy(
                        buf, out_hbm.at[pl.ds(0, t)], wsem[b]
                    ).wait()

                @pl.loop(0, t, step=4)
                def _(r, _buf=buf, _i=i):
                    ci = jnp.full((16,), _i, jnp.int32)
                    offs = []
                    for u in range(4):
                        tid = plsc.load_gather(
                            ids_v, [ci, jnp.full((16,), r + u, jnp.int32)]
                        )
                        offs.append(tid * d + lanes)
                    for c in range(0, d, 16):
                        vals = [
                            plsc.load_gather(tab_v, [offs[u] + c])
                            for u in range(4)
                        ]
                        for u in range(4):
                            _buf[r + u, pl.ds(c, 16)] = vals[u]

                pltpu.async_copy(
                    buf,
                    out_hbm.at[pl.ds((wid * rows_w + i) * t, t)],
                    wsem[b],
                )

        for b in range(2):
            pltpu.make_async_copy(
                bufs[b], out_hbm.at[pl.ds(0, t)], wsem[b]
            ).wait()

    return k(comb_flat, token_ids)


def kernel(token_ids, coords, special_table, type_table, W, b):
    bsz, t = token_ids.shape
    d = special_table.shape[1]

    b2d = b.reshape(1, d)
    comb = _combined_table(special_table, type_table)
    special_flat = _sc_special(
        comb.reshape(3 * d), token_ids.astype(jnp.int32), d
    )
    coord_out = _coord_linear(
        coords[:, :, 0], coords[:, :, 1], W, b2d, type_table
    )
    return special_flat.reshape(bsz, t, d), coord_out
